# Initial kernel scaffold; baseline (speedup 1.0000x reference)
import jax
import jax.numpy as jnp
from jax.experimental import pallas as pl


def kernel(x, disaster_type, severity, location, params):
    raise NotImplementedError("write your pallas kernel here")


# speedup vs baseline: 2.3631x; 2.3631x over previous
import jax
import jax.numpy as jnp
from jax.experimental import pallas as pl
from jax.experimental.pallas import tpu as pltpu

_N_EXPERTS = 5
_EXPERT_OUT_DIMS = [4, 3, 2, 10, 1]
_DPAD = 16
_T = 2048

_f32 = jnp.float32
_bf16 = jnp.bfloat16


def _dot(a, b):
    return jnp.dot(a.astype(_bf16), b, preferred_element_type=_f32)


def _ln(h, g, b):
    m = jnp.mean(h, axis=-1, keepdims=True)
    v = jnp.mean((h - m) ** 2, axis=-1, keepdims=True)
    return (h - m) / jnp.sqrt(v + 1e-5) * g + b


def _gelu(x):
    return 0.5 * x * (1.0 + jax.lax.erf(x * 0.7071067811865476))


def _body(x_ref, dt_ref, sl_ref,
          encW1, encb1, encg1, encbe1, encW2, encb2,
          gfW, gfb, gfg, gfbe,
          embT, gmWt, gmWsl, gmb, gmg, gmbe,
          ggW1a, ggW1b, ggb1, ggW2, ggb2,
          eW1, eb1, eg1, ebe1, eW2, eb2, eg2, ebe2, eW3, eb3, ehW, ehb,
          out_ref, usage_ref):
    T = x_ref.shape[0]

    h = _dot(x_ref[...], encW1[...]) + encb1[...]
    h = _gelu(_ln(h, encg1[...], encbe1[...]))
    encoded = _dot(h, encW2[...]) + encb2[...]

    feat = _gelu(_ln(_dot(encoded, gfW[...]) + gfb[...],
                     gfg[...], gfbe[...]))
    oh = (dt_ref[...] == jax.lax.broadcasted_iota(jnp.int32, (T, 10), 1)
          ).astype(_bf16)
    temb = jnp.dot(oh, embT[...], preferred_element_type=_f32)
    meta_pre = (_dot(temb, gmWt[...]) + _dot(sl_ref[...], gmWsl[...])
                + gmb[...])
    meta_enc = _gelu(_ln(meta_pre, gmg[...], gmbe[...]))
    gate_h = _gelu(_dot(feat, ggW1a[...]) + _dot(meta_enc, ggW1b[...])
                   + ggb1[...])
    gl = _dot(gate_h, ggW2[...]) + ggb2[...]

    iota5 = jax.lax.broadcasted_iota(jnp.int32, (T, _N_EXPERTS), 1)
    m1 = jnp.max(gl, axis=-1, keepdims=True)
    i1 = jnp.min(jnp.where(gl == m1, iota5, 127), axis=-1, keepdims=True)
    mask1 = iota5 == i1
    gl2 = jnp.where(mask1, -jnp.inf, gl)
    m2 = jnp.max(gl2, axis=-1, keepdims=True)
    i2 = jnp.min(jnp.where(gl2 == m2, iota5, 127), axis=-1, keepdims=True)
    a = jnp.exp(m2 - m1)
    s = 1.0 + a
    p1 = 1.0 / s
    p2 = a / s

    e = jnp.exp(gl - m1)
    probs = e / jnp.sum(e, axis=-1, keepdims=True)
    usage_ref[...] = jnp.sum(probs, axis=0, keepdims=True).reshape(1, 1, _N_EXPERTS)

    off = 0
    for i, d in enumerate(_EXPERT_OUT_DIMS):
        hh = _gelu(_ln(_dot(encoded, eW1[i]) + eb1[i], eg1[i], ebe1[i]))
        hh = _gelu(_ln(_dot(hh, eW2[i]) + eb2[i], eg2[i], ebe2[i]))
        o = (_dot(hh, eW3[i]) + eb3[i])[:, :d]
        if i in (0, 3):
            om = jnp.max(o, axis=-1, keepdims=True)
            oe = jnp.exp(o - om)
            o = oe / jnp.sum(oe, axis=-1, keepdims=True)
        elif i == 2:
            o = jax.nn.softplus(o)
        elif i == 4:
            o = jax.nn.sigmoid(o)
        r = (jnp.where(i1 == i, p1, 0.0) + jnp.where(i2 == i, p2, 0.0))
        w = o * r
        pred = _dot(w, ehW[i][:d, :d]) + ehb[i][0:1, :d]
        out_ref[:, off:off + d] = pred
        off += d


def kernel(x, disaster_type, severity, location, params):
    p = params
    B = x.shape[0]
    T = min(_T, B)
    n_blocks = B // T

    dt = disaster_type.reshape(B, 1).astype(jnp.int32)
    sl = jnp.concatenate([severity, location], axis=-1)

    def row(v):
        return v.reshape(1, -1).astype(_f32)

    def bw(a):
        return a.astype(_bf16)

    gm_W = p['gm_W']

    def pad2(a, r, c):
        return jnp.pad(a, ((0, r - a.shape[0]), (0, c - a.shape[1])))

    es = p['experts']
    eW1 = bw(jnp.stack([e['W1'] for e in es]))
    eb1 = jnp.stack([row(e['b1']) for e in es])
    eg1 = jnp.stack([row(e['g1']) for e in es])
    ebe1 = jnp.stack([row(e['be1']) for e in es])
    eW2 = bw(jnp.stack([e['W2'] for e in es]))
    eb2 = jnp.stack([row(e['b2']) for e in es])
    eg2 = jnp.stack([row(e['g2']) for e in es])
    ebe2 = jnp.stack([row(e['be2']) for e in es])
    eW3 = bw(jnp.stack([pad2(e['W3'], 128, _DPAD) for e in es]))
    eb3 = jnp.stack([pad2(row(e['b3']), 1, _DPAD) for e in es])
    ehW = bw(jnp.stack([pad2(e['hW'], _DPAD, _DPAD) for e in es]))
    ehb = jnp.stack([pad2(row(e['hb']), 1, _DPAD) for e in es])

    weights = [
        bw(p['enc_W1']), row(p['enc_b1']), row(p['enc_g1']),
        row(p['enc_be1']), bw(p['enc_W2']), row(p['enc_b2']),
        bw(p['gf_W']), row(p['gf_b']), row(p['gf_g']), row(p['gf_be']),
        bw(p['emb']), bw(gm_W[:16]), bw(gm_W[16:22]),
        row(p['gm_b']), row(p['gm_g']), row(p['gm_be']),
        bw(p['gg_W1'][:64]), bw(p['gg_W1'][64:]), row(p['gg_b1']),
        bw(p['gg_W2']), row(p['gg_b2']),
        eW1, eb1, eg1, ebe1, eW2, eb2, eg2, ebe2, eW3, eb3, ehW, ehb,
    ]

    def wspec(a):
        nd = a.ndim
        return pl.BlockSpec(a.shape, lambda i, _n=nd: (0,) * _n)

    in_specs = [
        pl.BlockSpec((T, 64), lambda i: (i, 0)),
        pl.BlockSpec((T, 1), lambda i: (i, 0)),
        pl.BlockSpec((T, 6), lambda i: (i, 0)),
    ] + [wspec(w) for w in weights]

    out_shape = [
        jax.ShapeDtypeStruct((B, 20), _f32),
        jax.ShapeDtypeStruct((n_blocks, 1, _N_EXPERTS), _f32),
    ]
    out_specs = [
        pl.BlockSpec((T, 20), lambda i: (i, 0)),
        pl.BlockSpec((1, 1, _N_EXPERTS), lambda i: (i, 0, 0)),
    ]

    preds, psums = pl.pallas_call(
        _body,
        grid=(n_blocks,),
        in_specs=in_specs,
        out_specs=out_specs,
        out_shape=out_shape,
        compiler_params=pltpu.CompilerParams(
            dimension_semantics=("parallel",)),
    )(x, dt, sl, *weights)

    usage = jnp.sum(psums.reshape(n_blocks, _N_EXPERTS), axis=0) / B
    t = 1.0 / _N_EXPERTS
    lb = jnp.sum(t * (jnp.log(t) - jnp.log(usage))) / _N_EXPERTS
    return preds, lb
